# baseline (speedup 1.0000x reference)
import functools

import jax
import jax.numpy as jnp
from jax import lax
from jax.experimental import pallas as pl
from jax.experimental.pallas import tpu as pltpu
from jax.experimental.pallas import tpu_sc as plsc

NC = 2
NS = 16
NW = NC * NS
CHUNK = 128
GROUP = 4
NBUF = 2
DEGW = 8

_MESH = plsc.VectorSubcoreMesh(
    core_axis_name="c", subcore_axis_name="s", num_cores=NC, num_subcores=NS
)
_SC_PARAMS = pltpu.CompilerParams(use_tc_tiling_on_sc=False,
                                  needs_layout_passes=False)


def _load_idx(ei3, row, pad2d, buf, wid, n_chunks, real_chunks):
    arr2d = ei3.at[row]
    last = NW - 1
    n_real_last = real_chunks - last * n_chunks

    @pl.when(wid < last)
    def _():
        pltpu.sync_copy(arr2d.at[pl.ds(wid * n_chunks, n_chunks)], buf)

    @pl.when(wid == last)
    def _():
        pltpu.sync_copy(arr2d.at[pl.ds(last * n_chunks, n_real_last)],
                        buf.at[pl.ds(0, n_real_last)])
        pltpu.sync_copy(pad2d, buf.at[pl.ds(n_real_last,
                                            n_chunks - n_real_last)])


def _deg_body(ei3, pad2d, zc, e0_hbm, out, e0, sidx, didx,
              acc_s, acc_d, sem):
    c = lax.axis_index("c")
    s = lax.axis_index("s")
    wid = s * NC + c
    n_chunks = sidx.shape[0]
    real_chunks = ei3.shape[1]
    bpt = acc_s.shape[0] // NS

    pltpu.sync_copy(e0_hbm, e0)
    pltpu.sync_copy(zc, acc_s.at[pl.ds(s * bpt, bpt)])
    pltpu.sync_copy(zc, acc_d.at[pl.ds(s * bpt, bpt)])
    _load_idx(ei3, 0, pad2d, sidx, wid, n_chunks, real_chunks)
    _load_idx(ei3, 1, pad2d, didx, wid, n_chunks, real_chunks)
    plsc.subcore_barrier()

    @pl.loop(0, n_chunks, step=GROUP)
    def _(g):
        descs = [
            pltpu.async_copy(e0, acc_s.at[sidx.at[g + j]], sem, add=True)
            for j in range(GROUP)
        ] + [
            pltpu.async_copy(e0, acc_d.at[didx.at[g + j]], sem, add=True)
            for j in range(GROUP)
        ]
        for d in descs:
            d.wait()

    plsc.subcore_barrier()
    pltpu.sync_copy(acc_s.at[pl.ds(s * bpt, bpt)],
                    out.at[0].at[c].at[pl.ds(s * bpt, bpt)])
    pltpu.sync_copy(acc_d.at[pl.ds(s * bpt, bpt)],
                    out.at[1].at[c].at[pl.ds(s * bpt, bpt)])


def _deg_call(ei3, pad2d, nbins, n_chunks):
    return pl.kernel(
        _deg_body,
        out_type=jax.ShapeDtypeStruct((2, NC, nbins, DEGW), jnp.float32),
        mesh=_MESH,
        compiler_params=_SC_PARAMS,
        scratch_types=[
            pltpu.VMEM((CHUNK, DEGW), jnp.float32),
            pltpu.VMEM((n_chunks, CHUNK), jnp.int32),
            pltpu.VMEM((n_chunks, CHUNK), jnp.int32),
            pltpu.VMEM_SHARED((nbins, DEGW), jnp.float32),
            pltpu.VMEM_SHARED((nbins, DEGW), jnp.float32),
            pltpu.SemaphoreType.DMA,
        ],
    )(ei3, pad2d,
      jnp.zeros((nbins // NS, DEGW), jnp.float32),
      jnp.zeros((CHUNK, DEGW), jnp.float32).at[:, 0].set(1.0))


def _agg_body(h, ei3, pad2d, zrows, out, sidx, didx, rows, acc,
              gsem, ssem):
    c = lax.axis_index("c")
    s = lax.axis_index("s")
    wid = s * NC + c
    n_chunks = sidx.shape[0]
    real_chunks = ei3.shape[1]
    n_groups = n_chunks // GROUP
    rows_per_tile = acc.shape[0] // NS

    pltpu.sync_copy(zrows, acc.at[pl.ds(s * rows_per_tile, rows_per_tile)])
    _load_idx(ei3, 0, pad2d, sidx, wid, n_chunks, real_chunks)
    _load_idx(ei3, 1, pad2d, didx, wid, n_chunks, real_chunks)
    plsc.subcore_barrier()

    for j in range(GROUP):
        pltpu.async_copy(h.at[sidx.at[j]], rows.at[0, j], gsem)

    @pl.loop(0, n_groups, step=NBUF)
    def _(g):
        for p in range(NBUF):
            gg = g + p
            q = (p + 1) % NBUF
            @pl.when(gg > 0)
            def _():
                for j in range(GROUP):
                    pltpu.make_async_copy(
                        rows.at[q, j], acc.at[didx.at[(gg - 1) * GROUP + j]],
                        ssem,
                    ).wait()
            for j in range(GROUP):
                pltpu.make_async_copy(
                    h.at[sidx.at[gg * GROUP + j]], rows.at[p, j], gsem
                ).wait()
            @pl.when(gg + 1 < n_groups)
            def _():
                for j in range(GROUP):
                    pltpu.async_copy(
                        h.at[sidx.at[(gg + 1) * GROUP + j]],
                        rows.at[q, j], gsem,
                    )
            for j in range(GROUP):
                pltpu.async_copy(rows.at[p, j], acc.at[didx.at[gg * GROUP + j]],
                                 ssem, add=True)

    for j in range(GROUP):
        pltpu.make_async_copy(
            rows.at[(n_groups - 1) % NBUF, j],
            acc.at[didx.at[(n_groups - 1) * GROUP + j]], ssem,
        ).wait()

    plsc.subcore_barrier()
    pltpu.sync_copy(
        acc.at[pl.ds(s * rows_per_tile, rows_per_tile)],
        out.at[c].at[pl.ds(s * rows_per_tile, rows_per_tile)],
    )


def _agg_call(h, ei3, pad2d, n_chunks):
    nr, hdim = h.shape
    return pl.kernel(
        _agg_body,
        out_type=jax.ShapeDtypeStruct((NC, nr, hdim), jnp.float32),
        mesh=_MESH,
        compiler_params=_SC_PARAMS,
        scratch_types=[
            pltpu.VMEM((n_chunks, CHUNK), jnp.int32),
            pltpu.VMEM((n_chunks, CHUNK), jnp.int32),
            pltpu.VMEM((NBUF, GROUP, CHUNK, hdim), jnp.float32),
            pltpu.VMEM_SHARED((nr, hdim), jnp.float32),
            pltpu.SemaphoreType.DMA,
            pltpu.SemaphoreType.DMA,
        ],
    )(h, ei3, pad2d, jnp.zeros((nr // NS, hdim), jnp.float32))


def _mm_body(x_ref, w_ref, o_ref):
    o_ref[...] = jnp.dot(x_ref[...], w_ref[...],
                         preferred_element_type=jnp.float32)


def _scale_body(deg_ref, g_ref, h_ref):
    cnt = deg_ref[:, 0] + deg_ref[:, 1]
    rso = lax.rsqrt(jnp.maximum(cnt[0, :, 0:1], 1.0))
    h_ref[...] = g_ref[...] * rso


def _layer2_body(deg_ref, p_ref, b1_ref, w2_ref, h2_ref):
    cnt = deg_ref[:, 0] + deg_ref[:, 1]
    rso = lax.rsqrt(jnp.maximum(cnt[0, :, 0:1], 1.0))
    rsi = lax.rsqrt(jnp.maximum(cnt[1, :, 0:1], 1.0))
    agg = (p_ref[0] + p_ref[1]) * rsi + b1_ref[...]
    x1 = jnp.maximum(agg, 0.0)
    h2_ref[...] = jnp.dot(x1 * rso, w2_ref[...],
                          preferred_element_type=jnp.float32)


def _emb_body(deg_ref, q_ref, b2_ref, emb_ref, emb2_ref):
    cnt = deg_ref[:, 0] + deg_ref[:, 1]
    rsi = lax.rsqrt(jnp.maximum(cnt[1, :, 0:1], 1.0))
    emb = (q_ref[0] + q_ref[1]) * rsi + b2_ref[...]
    emb_ref[...] = emb
    emb2_ref[...] = emb


def _logits_body(a_ref, b_ref, o_ref):
    acc = lax.dot_general(a_ref[...], b_ref[...],
                          (((1,), (1,)), ((), ())),
                          preferred_element_type=jnp.float32)
    o_ref[...] = jax.nn.sigmoid(acc)


def kernel(features, edge_index, W1, b1, W2, b2):
    n, in_dim = features.shape
    e = edge_index.shape[1]
    h1d = W1.shape[1]
    h2d = W2.shape[1]

    nr = ((n + CHUNK) + NW * 16 - 1) // (NW * 16) * (NW * 16)
    e_pad = (e + NW * GROUP * CHUNK - 1) // (NW * GROUP * CHUNK) * (NW * GROUP * CHUNK)
    n_chunks = e_pad // (NW * CHUNK)

    assert e % CHUNK == 0 and e // CHUNK >= (NW - 1) * n_chunks
    ei3 = edge_index.reshape(2, -1, CHUNK)
    pad_chunks = NW * n_chunks - e // CHUNK
    pad2d = jnp.broadcast_to(
        (jnp.arange(CHUNK, dtype=jnp.int32) + n)[None, :],
        (max(pad_chunks, 1), CHUNK))

    x_pad = jnp.pad(features, ((0, nr - n), (0, 0)))

    deg = _deg_call(ei3, pad2d, nr, n_chunks)

    blk = 2048
    g1 = pl.pallas_call(
        _mm_body,
        grid=(nr // blk,),
        in_specs=[pl.BlockSpec((blk, in_dim), lambda i: (i, 0)),
                  pl.BlockSpec((in_dim, h1d), lambda i: (0, 0))],
        out_specs=pl.BlockSpec((blk, h1d), lambda i: (i, 0)),
        out_shape=jax.ShapeDtypeStruct((nr, h1d), jnp.float32),
    )(x_pad, W1)

    deg_spec = lambda b: pl.BlockSpec((2, NC, b, DEGW), lambda i: (0, 0, i, 0))
    h1 = pl.pallas_call(
        _scale_body,
        grid=(nr // blk,),
        in_specs=[deg_spec(blk),
                  pl.BlockSpec((blk, h1d), lambda i: (i, 0))],
        out_specs=pl.BlockSpec((blk, h1d), lambda i: (i, 0)),
        out_shape=jax.ShapeDtypeStruct((nr, h1d), jnp.float32),
    )(deg, g1)

    p1 = _agg_call(h1, ei3, pad2d, n_chunks)

    h2 = pl.pallas_call(
        _layer2_body,
        grid=(nr // blk,),
        in_specs=[deg_spec(blk),
                  pl.BlockSpec((2, blk, h1d), lambda i: (0, i, 0)),
                  pl.BlockSpec((h1d,), lambda i: (0,)),
                  pl.BlockSpec((h1d, h2d), lambda i: (0, 0))],
        out_specs=pl.BlockSpec((blk, h2d), lambda i: (i, 0)),
        out_shape=jax.ShapeDtypeStruct((nr, h2d), jnp.float32),
    )(deg, p1, b1, W2)

    p2 = _agg_call(h2, ei3, pad2d, n_chunks)

    emb, emb2 = pl.pallas_call(
        _emb_body,
        grid=(n // 2000,),
        in_specs=[deg_spec(2000),
                  pl.BlockSpec((2, 2000, h2d), lambda i: (0, i, 0)),
                  pl.BlockSpec((h2d,), lambda i: (0,))],
        out_specs=[pl.BlockSpec((2000, h2d), lambda i: (i, 0)),
                   pl.BlockSpec((2000, h2d), lambda i: (i, 0))],
        out_shape=[jax.ShapeDtypeStruct((n, h2d), jnp.float32),
                   jax.ShapeDtypeStruct((n, h2d), jnp.float32)],
    )(deg, p2, b2)

    bm = bn = 2048
    grid = (pl.cdiv(n, bm), pl.cdiv(n, bn))
    logits = pl.pallas_call(
        _logits_body,
        grid=grid,
        in_specs=[
            pl.BlockSpec((bm, h2d), lambda i, j: (i, 0)),
            pl.BlockSpec((bn, h2d), lambda i, j: (j, 0)),
        ],
        out_specs=pl.BlockSpec((bm, bn), lambda i, j: (i, j)),
        out_shape=jax.ShapeDtypeStruct((n, n), jnp.float32),
    )(emb2, emb2)

    return (emb, logits)


# speedup vs baseline: 1.0046x; 1.0046x over previous
import functools

import jax
import jax.numpy as jnp
from jax import lax
from jax.experimental import pallas as pl
from jax.experimental.pallas import tpu as pltpu
from jax.experimental.pallas import tpu_sc as plsc

NC = 2
NS = 16
NW = NC * NS
CHUNK = 128
GROUP = 4
NBUF = 2
DEGW = 8

_MESH = plsc.VectorSubcoreMesh(
    core_axis_name="c", subcore_axis_name="s", num_cores=NC, num_subcores=NS
)
_SC_PARAMS = pltpu.CompilerParams(use_tc_tiling_on_sc=False,
                                  needs_layout_passes=False)


def _load_idx(ei3, row, pad2d, buf, wid, n_chunks, real_chunks):
    arr2d = ei3.at[row]
    last = NW - 1
    n_real_last = real_chunks - last * n_chunks

    @pl.when(wid < last)
    def _():
        pltpu.sync_copy(arr2d.at[pl.ds(wid * n_chunks, n_chunks)], buf)

    @pl.when(wid == last)
    def _():
        pltpu.sync_copy(arr2d.at[pl.ds(last * n_chunks, n_real_last)],
                        buf.at[pl.ds(0, n_real_last)])
        pltpu.sync_copy(pad2d, buf.at[pl.ds(n_real_last,
                                            n_chunks - n_real_last)])


def _deg_body(ei3, pad2d, zc, e0_hbm, out, e0, sidx, didx,
              acc_s, acc_d, sem):
    c = lax.axis_index("c")
    s = lax.axis_index("s")
    wid = s * NC + c
    n_chunks = sidx.shape[0]
    real_chunks = ei3.shape[1]
    bpt = acc_s.shape[0] // NS

    pltpu.sync_copy(e0_hbm, e0)
    pltpu.sync_copy(zc, acc_s.at[pl.ds(s * bpt, bpt)])
    pltpu.sync_copy(zc, acc_d.at[pl.ds(s * bpt, bpt)])
    _load_idx(ei3, 0, pad2d, sidx, wid, n_chunks, real_chunks)
    _load_idx(ei3, 1, pad2d, didx, wid, n_chunks, real_chunks)
    plsc.subcore_barrier()

    @pl.loop(0, n_chunks, step=GROUP)
    def _(g):
        descs = [
            pltpu.async_copy(e0, acc_s.at[sidx.at[g + j]], sem, add=True)
            for j in range(GROUP)
        ] + [
            pltpu.async_copy(e0, acc_d.at[didx.at[g + j]], sem, add=True)
            for j in range(GROUP)
        ]
        for d in descs:
            d.wait()

    plsc.subcore_barrier()
    pltpu.sync_copy(acc_s.at[pl.ds(s * bpt, bpt)],
                    out.at[0].at[c].at[pl.ds(s * bpt, bpt)])
    pltpu.sync_copy(acc_d.at[pl.ds(s * bpt, bpt)],
                    out.at[1].at[c].at[pl.ds(s * bpt, bpt)])


def _deg_call(ei3, pad2d, nbins, n_chunks):
    return pl.kernel(
        _deg_body,
        out_type=jax.ShapeDtypeStruct((2, NC, nbins, DEGW), jnp.float32),
        mesh=_MESH,
        compiler_params=_SC_PARAMS,
        scratch_types=[
            pltpu.VMEM((CHUNK, DEGW), jnp.float32),
            pltpu.VMEM((n_chunks, CHUNK), jnp.int32),
            pltpu.VMEM((n_chunks, CHUNK), jnp.int32),
            pltpu.VMEM_SHARED((nbins, DEGW), jnp.float32),
            pltpu.VMEM_SHARED((nbins, DEGW), jnp.float32),
            pltpu.SemaphoreType.DMA,
        ],
    )(ei3, pad2d,
      jnp.zeros((nbins // NS, DEGW), jnp.float32),
      jnp.zeros((CHUNK, DEGW), jnp.float32).at[:, 0].set(1.0))


def _agg_body(h, ei3, pad2d, zrows, out, sidx, didx, rows, acc,
              gsem, ssem):
    c = lax.axis_index("c")
    s = lax.axis_index("s")
    wid = s * NC + c
    n_chunks = sidx.shape[0]
    real_chunks = ei3.shape[1]
    n_groups = n_chunks // GROUP
    rows_per_tile = acc.shape[0] // NS

    pltpu.sync_copy(zrows, acc.at[pl.ds(s * rows_per_tile, rows_per_tile)])
    _load_idx(ei3, 0, pad2d, sidx, wid, n_chunks, real_chunks)
    _load_idx(ei3, 1, pad2d, didx, wid, n_chunks, real_chunks)
    plsc.subcore_barrier()

    for j in range(GROUP):
        pltpu.async_copy(h.at[sidx.at[j]], rows.at[0, j], gsem)

    @pl.loop(0, n_groups, step=NBUF)
    def _(g):
        for p in range(NBUF):
            gg = g + p
            q = (p + 1) % NBUF
            @pl.when(gg > 0)
            def _():
                for j in range(GROUP):
                    pltpu.make_async_copy(
                        rows.at[q, j], acc.at[didx.at[(gg - 1) * GROUP + j]],
                        ssem,
                    ).wait()
            for j in range(GROUP):
                pltpu.make_async_copy(
                    h.at[sidx.at[gg * GROUP + j]], rows.at[p, j], gsem
                ).wait()
            @pl.when(gg + 1 < n_groups)
            def _():
                for j in range(GROUP):
                    pltpu.async_copy(
                        h.at[sidx.at[(gg + 1) * GROUP + j]],
                        rows.at[q, j], gsem,
                    )
            for j in range(GROUP):
                pltpu.async_copy(rows.at[p, j], acc.at[didx.at[gg * GROUP + j]],
                                 ssem, add=True)

    for j in range(GROUP):
        pltpu.make_async_copy(
            rows.at[(n_groups - 1) % NBUF, j],
            acc.at[didx.at[(n_groups - 1) * GROUP + j]], ssem,
        ).wait()

    plsc.subcore_barrier()
    pltpu.sync_copy(
        acc.at[pl.ds(s * rows_per_tile, rows_per_tile)],
        out.at[c].at[pl.ds(s * rows_per_tile, rows_per_tile)],
    )


def _agg_call(h, ei3, pad2d, n_chunks):
    nr, hdim = h.shape
    return pl.kernel(
        _agg_body,
        out_type=jax.ShapeDtypeStruct((NC, nr, hdim), jnp.float32),
        mesh=_MESH,
        compiler_params=_SC_PARAMS,
        scratch_types=[
            pltpu.VMEM((n_chunks, CHUNK), jnp.int32),
            pltpu.VMEM((n_chunks, CHUNK), jnp.int32),
            pltpu.VMEM((NBUF, GROUP, CHUNK, hdim), jnp.float32),
            pltpu.VMEM_SHARED((nr, hdim), jnp.float32),
            pltpu.SemaphoreType.DMA,
            pltpu.SemaphoreType.DMA,
        ],
    )(h, ei3, pad2d, jnp.zeros((nr // NS, hdim), jnp.float32))


def _mm_body(x_ref, w_ref, o_ref):
    o_ref[...] = jnp.dot(x_ref[...], w_ref[...],
                         preferred_element_type=jnp.float32)


def _scale_body(deg_ref, g_ref, h_ref):
    cnt = deg_ref[:, 0] + deg_ref[:, 1]
    rso = lax.rsqrt(jnp.maximum(cnt[0, :, 0:1], 1.0))
    h_ref[...] = g_ref[...] * rso


def _layer2_body(deg_ref, p_ref, b1_ref, w2_ref, h2_ref):
    cnt = deg_ref[:, 0] + deg_ref[:, 1]
    rso = lax.rsqrt(jnp.maximum(cnt[0, :, 0:1], 1.0))
    rsi = lax.rsqrt(jnp.maximum(cnt[1, :, 0:1], 1.0))
    agg = (p_ref[0] + p_ref[1]) * rsi + b1_ref[...]
    x1 = jnp.maximum(agg, 0.0)
    h2_ref[...] = jnp.dot(x1 * rso, w2_ref[...],
                          preferred_element_type=jnp.float32)


def _emb_body(deg_ref, q_ref, b2_ref, emb_ref):
    cnt = deg_ref[:, 0] + deg_ref[:, 1]
    rsi = lax.rsqrt(jnp.maximum(cnt[1, :, 0:1], 1.0))
    emb_ref[...] = (q_ref[0] + q_ref[1]) * rsi + b2_ref[...]


def _logits_body(a_ref, b_ref, o_ref):
    acc = lax.dot_general(a_ref[...], b_ref[...],
                          (((1,), (1,)), ((), ())),
                          preferred_element_type=jnp.float32)
    o_ref[...] = jax.nn.sigmoid(acc)


def kernel(features, edge_index, W1, b1, W2, b2):
    n, in_dim = features.shape
    e = edge_index.shape[1]
    h1d = W1.shape[1]
    h2d = W2.shape[1]

    nr = ((n + CHUNK) + NW * 16 - 1) // (NW * 16) * (NW * 16)
    e_pad = (e + NW * GROUP * CHUNK - 1) // (NW * GROUP * CHUNK) * (NW * GROUP * CHUNK)
    n_chunks = e_pad // (NW * CHUNK)

    assert e % CHUNK == 0 and e // CHUNK >= (NW - 1) * n_chunks
    ei3 = edge_index.reshape(2, -1, CHUNK)
    pad_chunks = NW * n_chunks - e // CHUNK
    pad2d = jnp.broadcast_to(
        (jnp.arange(CHUNK, dtype=jnp.int32) + n)[None, :],
        (max(pad_chunks, 1), CHUNK))

    x_pad = jnp.pad(features, ((0, nr - n), (0, 0)))

    deg = _deg_call(ei3, pad2d, nr, n_chunks)

    blk = 2048
    g1 = pl.pallas_call(
        _mm_body,
        grid=(nr // blk,),
        in_specs=[pl.BlockSpec((blk, in_dim), lambda i: (i, 0)),
                  pl.BlockSpec((in_dim, h1d), lambda i: (0, 0))],
        out_specs=pl.BlockSpec((blk, h1d), lambda i: (i, 0)),
        out_shape=jax.ShapeDtypeStruct((nr, h1d), jnp.float32),
    )(x_pad, W1)

    deg_spec = lambda b: pl.BlockSpec((2, NC, b, DEGW), lambda i: (0, 0, i, 0))
    h1 = pl.pallas_call(
        _scale_body,
        grid=(nr // blk,),
        in_specs=[deg_spec(blk),
                  pl.BlockSpec((blk, h1d), lambda i: (i, 0))],
        out_specs=pl.BlockSpec((blk, h1d), lambda i: (i, 0)),
        out_shape=jax.ShapeDtypeStruct((nr, h1d), jnp.float32),
    )(deg, g1)

    p1 = _agg_call(h1, ei3, pad2d, n_chunks)

    h2 = pl.pallas_call(
        _layer2_body,
        grid=(nr // blk,),
        in_specs=[deg_spec(blk),
                  pl.BlockSpec((2, blk, h1d), lambda i: (0, i, 0)),
                  pl.BlockSpec((h1d,), lambda i: (0,)),
                  pl.BlockSpec((h1d, h2d), lambda i: (0, 0))],
        out_specs=pl.BlockSpec((blk, h2d), lambda i: (i, 0)),
        out_shape=jax.ShapeDtypeStruct((nr, h2d), jnp.float32),
    )(deg, p1, b1, W2)

    p2 = _agg_call(h2, ei3, pad2d, n_chunks)

    emb = pl.pallas_call(
        _emb_body,
        grid=(n // 2000,),
        in_specs=[deg_spec(2000),
                  pl.BlockSpec((2, 2000, h2d), lambda i: (0, i, 0)),
                  pl.BlockSpec((h2d,), lambda i: (0,))],
        out_specs=pl.BlockSpec((2000, h2d), lambda i: (i, 0)),
        out_shape=jax.ShapeDtypeStruct((n, h2d), jnp.float32),
    )(deg, p2, b2)

    bm = bn = 2048
    grid = (pl.cdiv(n, bm), pl.cdiv(n, bn))
    logits = pl.pallas_call(
        _logits_body,
        grid=grid,
        in_specs=[
            pl.BlockSpec((bm, h2d), lambda i, j: (i, 0)),
            pl.BlockSpec((bn, h2d), lambda i, j: (j, 0)),
        ],
        out_specs=pl.BlockSpec((bm, bn), lambda i, j: (i, j)),
        out_shape=jax.ShapeDtypeStruct((n, n), jnp.float32),
    )(emb, emb)

    return (emb, logits)
